# baseline (speedup 1.0000x reference)
import functools

import jax
import jax.numpy as jnp
from jax import lax
from jax.experimental import pallas as pl
from jax.experimental.pallas import tpu as pltpu
from jax.experimental.pallas import tpu_sc as plsc

_NC = 2
_NS = 16
_NW = _NC * _NS
_L = 16
_D = 32
_R = 100
_T = 60000
_N = 1000000
_G = 128
_NG = 15
_PER_W = _G * _NG
_TPAD = _NW * _PER_W


def _body(hidx_hbm, ridx_hbm, tidx_hbm, node_hbm, rel_hbm, out_hbm,
          hidx_v, ridx_v, tidx_v, rel_v, hsup_v, tsup_v, hrows_v, trows_v,
          score_v, sem):
  wid = lax.axis_index("s") * _NC + lax.axis_index("c")
  base = wid * _PER_W

  pltpu.sync_copy(hidx_hbm.at[pl.ds(base, _PER_W)], hidx_v)
  pltpu.sync_copy(ridx_hbm.at[pl.ds(base, _PER_W)], ridx_v)
  pltpu.sync_copy(tidx_hbm.at[pl.ds(base, _PER_W)], tidx_v)
  pltpu.sync_copy(rel_hbm, rel_v)

  def group(j, carry):
    def sup(i, c):
      hseg = hidx_v[pl.ds(j * _G + i * _L, _L)]
      tseg = tidx_v[pl.ds(j * _G + i * _L, _L)]
      hsup_v[pl.ds(i * _L, _L)] = lax.shift_right_logical(hseg, 2)
      tsup_v[pl.ds(i * _L, _L)] = lax.shift_right_logical(tseg, 2)
      return c

    lax.fori_loop(0, _G // _L, sup, 0)

    pltpu.async_copy(node_hbm.at[hsup_v], hrows_v, sem).wait()
    pltpu.async_copy(node_hbm.at[tsup_v], trows_v, sem).wait()

    def step(k, c):
      rows = lax.iota(jnp.int32, _L) + k * _L
      hids = hidx_v[pl.ds(j * _G + k * _L, _L)]
      tids = tidx_v[pl.ds(j * _G + k * _L, _L)]
      hoff = (hids & 3) * _D
      toff = (tids & 3) * _D
      rbase = ridx_v[pl.ds(j * _G + k * _L, _L)] * _D
      acc = jnp.zeros((_L,), jnp.float32)
      for d in range(_D):
        hv = plsc.load_gather(hrows_v, [rows, hoff + d])
        tv = plsc.load_gather(trows_v, [rows, toff + d])
        rv = plsc.load_gather(rel_v, [rbase + d])
        acc = acc + hv * rv * tv
      score_v[pl.ds(j * _G + k * _L, _L)] = acc
      return c

    lax.fori_loop(0, _G // _L, step, 0)
    return carry

  lax.fori_loop(0, _NG, group, 0)
  pltpu.sync_copy(score_v, out_hbm.at[pl.ds(wid * _PER_W, _PER_W)])


@functools.partial(
    pl.kernel,
    out_type=jax.ShapeDtypeStruct((_TPAD,), jnp.float32),
    mesh=plsc.VectorSubcoreMesh(core_axis_name="c", subcore_axis_name="s"),
    compiler_params=pltpu.CompilerParams(needs_layout_passes=False),
    scratch_types=[
        pltpu.VMEM((_PER_W,), jnp.int32),
        pltpu.VMEM((_PER_W,), jnp.int32),
        pltpu.VMEM((_PER_W,), jnp.int32),
        pltpu.VMEM((_R * _D,), jnp.float32),
        pltpu.VMEM((_G,), jnp.int32),
        pltpu.VMEM((_G,), jnp.int32),
        pltpu.VMEM((_G, 128), jnp.float32),
        pltpu.VMEM((_G, 128), jnp.float32),
        pltpu.VMEM((_PER_W,), jnp.float32),
        pltpu.SemaphoreType.DMA,
    ],
)
def _score_kernel(hidx, ridx, tidx, node, rel, out, *scratch):
  _body(hidx, ridx, tidx, node, rel, out, *scratch)


def kernel(h_idx, r_idx, t_idx, node_embedding, relational_embedding):
  pad = _TPAD - _T
  zpad = jnp.zeros((pad,), jnp.int32)
  h2 = jnp.concatenate([h_idx, zpad])
  r2 = jnp.concatenate([r_idx, zpad])
  t2 = jnp.concatenate([t_idx, zpad])
  node128 = node_embedding.reshape(_N * _D // 128, 128)
  rel_flat = relational_embedding.reshape(_R * _D)
  score = _score_kernel(h2, r2, t2, node128, rel_flat)
  return score[:_T]


# speedup vs baseline: 3.4909x; 3.4909x over previous
import functools

import jax
import jax.numpy as jnp
from jax import lax
from jax.experimental import pallas as pl
from jax.experimental.pallas import tpu as pltpu
from jax.experimental.pallas import tpu_sc as plsc

_NC = 2
_NS = 16
_L = 16
_D = 32
_DC = _D // _NC
_R = 100
_T = 60000
_N = 1000000
_TPAD = 61440
_PW = _TPAD // _NS
_NCH = _PW // 128


def _sweep_body(hidx_hbm, ridx_hbm, tidx_hbm, nodet_hbm, relt_hbm, part_hbm,
                hidx_v, ridx_v, tidx_v, rel_v, hbuf_v, tbuf_v, acc_v,
                shared_v, sem):
  cid = lax.axis_index("c")
  sid = lax.axis_index("s")
  base = sid * _PW

  pltpu.sync_copy(hidx_hbm.at[pl.ds(base, _PW)], hidx_v)
  pltpu.sync_copy(ridx_hbm.at[pl.ds(base, _PW)], ridx_v)
  pltpu.sync_copy(tidx_hbm.at[pl.ds(base, _PW)], tidx_v)
  pltpu.sync_copy(relt_hbm, rel_v)

  def zero(k, c):
    acc_v[pl.ds(k * _L, _L)] = jnp.zeros((_L,), jnp.float32)
    return c

  lax.fori_loop(0, _PW // _L, zero, 0)

  def dim_step(d, carry):
    dim = cid * _DC + d

    @pl.when(sid == 0)
    def _():
      pltpu.sync_copy(nodet_hbm.at[dim], shared_v)

    plsc.subcore_barrier()

    copies = []
    for j in range(_NCH):
      copies.append(pltpu.async_copy(
          shared_v.at[hidx_v.at[pl.ds(j * 128, 128)]],
          hbuf_v.at[pl.ds(j * 128, 128)], sem))
      copies.append(pltpu.async_copy(
          shared_v.at[tidx_v.at[pl.ds(j * 128, 128)]],
          tbuf_v.at[pl.ds(j * 128, 128)], sem))
    for c in copies:
      c.wait()

    roff = dim * _R

    def fma(k, c):
      sl = pl.ds(k * _L, _L)
      hv = hbuf_v[sl]
      tv = tbuf_v[sl]
      rv = plsc.load_gather(rel_v, [ridx_v[sl] + roff])
      acc_v[sl] = acc_v[sl] + hv * rv * tv
      return c

    lax.fori_loop(0, _PW // _L, fma, 0)
    plsc.subcore_barrier()
    return carry

  lax.fori_loop(0, _DC, dim_step, 0)
  pltpu.sync_copy(acc_v, part_hbm.at[pl.ds(cid * _TPAD + base, _PW)])


@functools.partial(
    pl.kernel,
    out_type=jax.ShapeDtypeStruct((_NC * _TPAD,), jnp.float32),
    mesh=plsc.VectorSubcoreMesh(core_axis_name="c", subcore_axis_name="s"),
    compiler_params=pltpu.CompilerParams(needs_layout_passes=False),
    scratch_types=[
        pltpu.VMEM((_PW,), jnp.int32),
        pltpu.VMEM((_PW,), jnp.int32),
        pltpu.VMEM((_PW,), jnp.int32),
        pltpu.VMEM((_R * _D,), jnp.float32),
        pltpu.VMEM((_PW,), jnp.float32),
        pltpu.VMEM((_PW,), jnp.float32),
        pltpu.VMEM((_PW,), jnp.float32),
        pltpu.VMEM_SHARED((_N,), jnp.float32),
        pltpu.SemaphoreType.DMA,
    ],
)
def _sweep_kernel(hidx, ridx, tidx, nodet, relt, part, *scratch):
  _sweep_body(hidx, ridx, tidx, nodet, relt, part, *scratch)


def _add_body(part_hbm, out_hbm, a_v, b_v):
  wid = lax.axis_index("s") * _NC + lax.axis_index("c")
  base = wid * (_TPAD // (_NC * _NS))
  n = _TPAD // (_NC * _NS)
  pltpu.sync_copy(part_hbm.at[pl.ds(base, n)], a_v)
  pltpu.sync_copy(part_hbm.at[pl.ds(_TPAD + base, n)], b_v)

  def add(k, c):
    sl = pl.ds(k * _L, _L)
    a_v[sl] = a_v[sl] + b_v[sl]
    return c

  lax.fori_loop(0, n // _L, add, 0)
  pltpu.sync_copy(a_v, out_hbm.at[pl.ds(base, n)])


@functools.partial(
    pl.kernel,
    out_type=jax.ShapeDtypeStruct((_TPAD,), jnp.float32),
    mesh=plsc.VectorSubcoreMesh(core_axis_name="c", subcore_axis_name="s"),
    compiler_params=pltpu.CompilerParams(needs_layout_passes=False),
    scratch_types=[
        pltpu.VMEM((_TPAD // (_NC * _NS),), jnp.float32),
        pltpu.VMEM((_TPAD // (_NC * _NS),), jnp.float32),
    ],
)
def _combine_kernel(part, out, *scratch):
  _add_body(part, out, *scratch)


def kernel(h_idx, r_idx, t_idx, node_embedding, relational_embedding):
  pad = _TPAD - _T
  zpad = jnp.zeros((pad,), jnp.int32)
  h2 = jnp.concatenate([h_idx, zpad])
  r2 = jnp.concatenate([r_idx, zpad])
  t2 = jnp.concatenate([t_idx, zpad])
  nodet = node_embedding.T
  relt = relational_embedding.T.reshape(_R * _D)
  part = _sweep_kernel(h2, r2, t2, nodet, relt)
  score = _combine_kernel(part)
  return score[:_T]


# speedup vs baseline: 3.8807x; 1.1117x over previous
import functools

import jax
import jax.numpy as jnp
from jax import lax
from jax.experimental import pallas as pl
from jax.experimental.pallas import tpu as pltpu
from jax.experimental.pallas import tpu_sc as plsc

_NC = 2
_NS = 16
_L = 16
_D = 32
_DC = _D // _NC
_R = 100
_T = 60000
_N = 1000000
_TPAD = 61440
_PW = _TPAD // _NS
_NCH = _PW // 128


def _sweep_body(hidx_hbm, ridx_hbm, tidx_hbm, nodet_hbm, relt_hbm, tail_hbm,
                part_hbm,
                hidx_v, ridx_v, tidx_v, rel_v, tail_v, hbuf_v, tbuf_v, acc_v,
                shared_v, sem):
  cid = lax.axis_index("c")
  sid = lax.axis_index("s")
  base = sid * _PW

  pltpu.sync_copy(hidx_hbm.at[pl.ds(base, _PW)], hidx_v)
  pltpu.sync_copy(ridx_hbm.at[pl.ds(base, _PW)], ridx_v)
  pltpu.sync_copy(tidx_hbm.at[pl.ds(base, _PW)], tidx_v)
  pltpu.sync_copy(relt_hbm, rel_v)
  pltpu.sync_copy(tail_hbm.at[pl.ds(cid * _DC * 64, _DC * 64)], tail_v)

  def zero(k, c):
    acc_v[pl.ds(k * _L, _L)] = jnp.zeros((_L,), jnp.float32)
    return c

  lax.fori_loop(0, _PW // _L, zero, 0)

  def dim_step(d, carry):
    dim = cid * _DC + d

    row = nodet_hbm.at[dim]
    for i in range(_NS - 1):
      @pl.when(sid == i)
      def _(i=i):
        pltpu.sync_copy(row.at[pl.ds(i * 66560, 66560)],
                        shared_v.at[pl.ds(i * 66560, 66560)])

    @pl.when(sid == _NS - 1)
    def _():
      pltpu.sync_copy(row.at[pl.ds(998400, 1536)],
                      shared_v.at[pl.ds(998400, 1536)])
      pltpu.sync_copy(tail_v.at[pl.ds(d * 64, 64)],
                      shared_v.at[pl.ds(999936, 64)])

    plsc.subcore_barrier()

    copies = []
    for j in range(_NCH):
      copies.append(pltpu.async_copy(
          shared_v.at[hidx_v.at[pl.ds(j * 128, 128)]],
          hbuf_v.at[pl.ds(j * 128, 128)], sem))
      copies.append(pltpu.async_copy(
          shared_v.at[tidx_v.at[pl.ds(j * 128, 128)]],
          tbuf_v.at[pl.ds(j * 128, 128)], sem))

    roff = dim * _R

    def fma(k, c):
      sl = pl.ds(k * _L, _L)
      hv = hbuf_v[sl]
      tv = tbuf_v[sl]
      rv = plsc.load_gather(rel_v, [ridx_v[sl] + roff])
      acc_v[sl] = acc_v[sl] + hv * rv * tv
      return c

    for j in range(_NCH):
      copies[2 * j].wait()
      copies[2 * j + 1].wait()
      lax.fori_loop(8 * j, 8 * (j + 1), fma, 0)

    plsc.subcore_barrier()
    return carry

  lax.fori_loop(0, _DC, dim_step, 0)
  pltpu.sync_copy(acc_v, part_hbm.at[pl.ds(cid * _TPAD + base, _PW)])


@functools.partial(
    pl.kernel,
    out_type=jax.ShapeDtypeStruct((_NC * _TPAD,), jnp.float32),
    mesh=plsc.VectorSubcoreMesh(core_axis_name="c", subcore_axis_name="s"),
    compiler_params=pltpu.CompilerParams(needs_layout_passes=False),
    scratch_types=[
        pltpu.VMEM((_PW,), jnp.int32),
        pltpu.VMEM((_PW,), jnp.int32),
        pltpu.VMEM((_PW,), jnp.int32),
        pltpu.VMEM((_R * _D,), jnp.float32),
        pltpu.VMEM((_DC * 64,), jnp.float32),
        pltpu.VMEM((_PW,), jnp.float32),
        pltpu.VMEM((_PW,), jnp.float32),
        pltpu.VMEM((_PW,), jnp.float32),
        pltpu.VMEM_SHARED((_N,), jnp.float32),
        pltpu.SemaphoreType.DMA,
    ],
)
def _sweep_kernel(hidx, ridx, tidx, nodet, relt, tail, part, *scratch):
  _sweep_body(hidx, ridx, tidx, nodet, relt, tail, part, *scratch)


def _add_body(part_hbm, out_hbm, a_v, b_v):
  wid = lax.axis_index("s") * _NC + lax.axis_index("c")
  base = wid * (_TPAD // (_NC * _NS))
  n = _TPAD // (_NC * _NS)
  pltpu.sync_copy(part_hbm.at[pl.ds(base, n)], a_v)
  pltpu.sync_copy(part_hbm.at[pl.ds(_TPAD + base, n)], b_v)

  def add(k, c):
    sl = pl.ds(k * _L, _L)
    a_v[sl] = a_v[sl] + b_v[sl]
    return c

  lax.fori_loop(0, n // _L, add, 0)
  pltpu.sync_copy(a_v, out_hbm.at[pl.ds(base, n)])


@functools.partial(
    pl.kernel,
    out_type=jax.ShapeDtypeStruct((_TPAD,), jnp.float32),
    mesh=plsc.VectorSubcoreMesh(core_axis_name="c", subcore_axis_name="s"),
    compiler_params=pltpu.CompilerParams(needs_layout_passes=False),
    scratch_types=[
        pltpu.VMEM((_TPAD // (_NC * _NS),), jnp.float32),
        pltpu.VMEM((_TPAD // (_NC * _NS),), jnp.float32),
    ],
)
def _combine_kernel(part, out, *scratch):
  _add_body(part, out, *scratch)


def kernel(h_idx, r_idx, t_idx, node_embedding, relational_embedding):
  pad = _TPAD - _T
  zpad = jnp.zeros((pad,), jnp.int32)
  h2 = jnp.concatenate([h_idx, zpad])
  r2 = jnp.concatenate([r_idx, zpad])
  t2 = jnp.concatenate([t_idx, zpad])
  nodet = node_embedding.T
  relt = relational_embedding.T.reshape(_R * _D)
  tail = node_embedding[_N - 64:].T.reshape(64 * _D)
  part = _sweep_kernel(h2, r2, t2, nodet, relt, tail)
  score = _combine_kernel(part)
  return score[:_T]
